# baseline (speedup 1.0000x reference)
import functools

import jax
import jax.numpy as jnp
from jax import lax
from jax.experimental import pallas as pl
from jax.experimental.pallas import tpu as pltpu
from jax.experimental.pallas import tpu_sc as plsc

N_SENT = 10000
N_DOC = 1000
NS_PAD = 10240
ND_PAD = 1024
D_IN = 768
D = 128
E_NEAR = 51200
E_IN = 25600
E_HAS = 25600

NC = 2
NT = 16
CH = 80

SENT_BLK = 512
DOC_BLK = 1024

SROWS = NS_PAD // NT
DROWS = ND_PAD // NT
ZCH = 64


def _prep_body(coeff1_ref, coeff2_ref, basis1_ref, loop1_ref, basis2_ref,
               loop2_ref, w_s1_ref, w_d1_ref, w_s2_ref, w_d2_ref):
    b1a = basis1_ref[0]
    b1b = basis1_ref[1]
    w10 = coeff1_ref[0, 0] * b1a + coeff1_ref[0, 1] * b1b
    w11 = coeff1_ref[1, 0] * b1a + coeff1_ref[1, 1] * b1b
    w12 = coeff1_ref[2, 0] * b1a + coeff1_ref[2, 1] * b1b
    w_s1_ref[...] = jnp.concatenate([w10, w11, loop1_ref[...]], axis=1)
    w_d1_ref[...] = jnp.concatenate([w12, loop1_ref[...]], axis=1)
    b2a = basis2_ref[0]
    b2b = basis2_ref[1]
    w20 = coeff2_ref[0, 0] * b2a + coeff2_ref[0, 1] * b2b
    w21 = coeff2_ref[1, 0] * b2a + coeff2_ref[1, 1] * b2b
    w22 = coeff2_ref[2, 0] * b2a + coeff2_ref[2, 1] * b2b
    w_s2_ref[...] = jnp.concatenate([w20, w21, loop2_ref[...]], axis=1)
    w_d2_ref[...] = jnp.concatenate([w22, loop2_ref[...]], axis=1)


def _prep_weights(coeff1, coeff2, basis1, loop_w1, basis2, loop_w2):
    smem = pl.BlockSpec(memory_space=pltpu.MemorySpace.SMEM)
    return pl.pallas_call(
        _prep_body,
        in_specs=[smem, smem, pl.BlockSpec((2, D_IN, D), lambda: (0, 0, 0)),
                  pl.BlockSpec((D_IN, D), lambda: (0, 0)),
                  pl.BlockSpec((2, D, D), lambda: (0, 0, 0)),
                  pl.BlockSpec((D, D), lambda: (0, 0))],
        out_specs=[pl.BlockSpec((D_IN, 3 * D), lambda: (0, 0)),
                   pl.BlockSpec((D_IN, 2 * D), lambda: (0, 0)),
                   pl.BlockSpec((D, 3 * D), lambda: (0, 0)),
                   pl.BlockSpec((D, 2 * D), lambda: (0, 0))],
        out_shape=[jax.ShapeDtypeStruct((D_IN, 3 * D), jnp.float32),
                   jax.ShapeDtypeStruct((D_IN, 2 * D), jnp.float32),
                   jax.ShapeDtypeStruct((D, 3 * D), jnp.float32),
                   jax.ShapeDtypeStruct((D, 2 * D), jnp.float32)],
    )(coeff1, coeff2, basis1, loop_w1, basis2, loop_w2)


def _proj_sent_body(x_ref, w_ref, t_a_ref, t_b_ref, self_ref):
    proj = jnp.dot(x_ref[...], w_ref[...], preferred_element_type=jnp.float32)
    t_a_ref[...] = proj[:, :D]
    t_b_ref[...] = proj[:, D:2 * D]
    self_ref[...] = proj[:, 2 * D:3 * D]


def _proj_sent(x, w, d_in):
    nblk = NS_PAD // SENT_BLK
    return pl.pallas_call(
        _proj_sent_body,
        grid=(nblk,),
        in_specs=[pl.BlockSpec((SENT_BLK, d_in), lambda i: (i, 0)),
                  pl.BlockSpec((d_in, 3 * D), lambda i: (0, 0))],
        out_specs=[pl.BlockSpec((SENT_BLK, D), lambda i: (i, 0)),
                   pl.BlockSpec((SENT_BLK, D), lambda i: (i, 0)),
                   pl.BlockSpec((SENT_BLK, D), lambda i: (i, 0))],
        out_shape=[jax.ShapeDtypeStruct((NS_PAD, D), jnp.float32),
                   jax.ShapeDtypeStruct((NS_PAD, D), jnp.float32),
                   jax.ShapeDtypeStruct((NS_PAD, D), jnp.float32)],
    )(x, w)


def _proj_doc_body(x_ref, w_ref, t_ref, self_ref):
    proj = jnp.dot(x_ref[...], w_ref[...], preferred_element_type=jnp.float32)
    t_ref[...] = proj[:, :D]
    self_ref[...] = proj[:, D:2 * D]


def _proj_doc(x, w, d_in):
    return pl.pallas_call(
        _proj_doc_body,
        grid=(ND_PAD // DOC_BLK,),
        in_specs=[pl.BlockSpec((DOC_BLK, d_in), lambda i: (i, 0)),
                  pl.BlockSpec((d_in, 2 * D), lambda i: (0, 0))],
        out_specs=[pl.BlockSpec((DOC_BLK, D), lambda i: (i, 0)),
                   pl.BlockSpec((DOC_BLK, D), lambda i: (i, 0))],
        out_shape=[jax.ShapeDtypeStruct((ND_PAD, D), jnp.float32),
                   jax.ShapeDtypeStruct((ND_PAD, D), jnp.float32)],
    )(x, w)


def _edge_loop(nchunks, tile_base, src_hbm, dst_hbm, table_hbm, acc_sh,
               deg_sh, sidx2, didx2, rows2, isem2, gsem2, ssem2, ones1d):
    def issue_idx(j, p):
        base = tile_base + j * CH
        pltpu.async_copy(src_hbm.at[pl.ds(base, CH)], sidx2[p], isem2[p])
        pltpu.async_copy(dst_hbm.at[pl.ds(base, CH)], didx2[p], isem2[p])

    def wait_idx(p):
        pltpu.make_async_copy(src_hbm.at[pl.ds(0, CH)], sidx2[p],
                              isem2[p]).wait()
        pltpu.make_async_copy(dst_hbm.at[pl.ds(0, CH)], didx2[p],
                              isem2[p]).wait()

    def issue_gather(p):
        pltpu.async_copy(table_hbm.at[sidx2[p]], rows2[p], gsem2[p])

    def wait_gather(p):
        pltpu.make_async_copy(table_hbm.at[sidx2[p]], rows2[p],
                              gsem2[p]).wait()

    def issue_scatter(p):
        pltpu.async_copy(rows2[p], acc_sh.at[didx2[p]], ssem2[p], add=True)
        pltpu.sync_copy(ones1d, deg_sh.at[didx2[p]], add=True)

    def wait_scatter(p):
        pltpu.make_async_copy(rows2[p], acc_sh.at[didx2[p]], ssem2[p]).wait()

    nk = nchunks // 2
    issue_idx(0, 0)

    def body(k, carry):
        j0 = 2 * k
        wait_idx(0)
        issue_gather(0)

        @pl.when(k > 0)
        def _():
            wait_scatter(1)
        issue_idx(j0 + 1, 1)
        wait_gather(0)
        issue_scatter(0)
        wait_idx(1)
        issue_gather(1)
        wait_scatter(0)

        @pl.when(k < nk - 1)
        def _():
            issue_idx(j0 + 2, 0)
        wait_gather(1)
        issue_scatter(1)
        return carry
    lax.fori_loop(0, nk, body, 0)
    wait_scatter(1)


def _norm_out(nchunks, row0, acc_sh, deg_sh, out_hbm, nbuf, degv, chrows):
    pltpu.sync_copy(deg_sh.at[pl.ds(row0, nchunks * chrows)],
                    degv.at[pl.ds(0, nchunks * chrows)])
    for q in range(nchunks):
        pltpu.sync_copy(acc_sh.at[pl.ds(row0 + q * chrows, chrows)],
                        nbuf.at[pl.ds(0, chrows)])

        def grp_body(g, carry):
            deg16 = degv[pl.ds(q * chrows + g * 16, 16)]
            rd16 = 1.0 / jnp.maximum(deg16, 1.0)
            for r in range(16):
                row = g * 16 + r
                rd = rd16[r]
                for k in range(D // 16):
                    nbuf[row, pl.ds(k * 16, 16)] = (
                        nbuf[row, pl.ds(k * 16, 16)] * rd)
            return carry
        lax.fori_loop(0, chrows // 16, grp_body, 0)
        pltpu.sync_copy(nbuf.at[pl.ds(0, chrows)],
                        out_hbm.at[pl.ds(row0 + q * chrows, chrows)])


def _sc_agg_body(t_near, t_in, t_has, near_src, near_dst, in_src, in_dst,
                 has_src, has_dst, out_near, out_in, out_has,
                 sidxa, sidxb, didxa, didxb, rowsa, rowsb, ones1d, degv,
                 isema, isemb, gsema, gsemb, ssema, ssemb,
                 acc_s, acc_d, deg_s, deg_d):
    c = lax.axis_index("c")
    s = lax.axis_index("s")
    sidx2 = (sidxa, sidxb)
    didx2 = (didxa, didxb)
    rows2 = (rowsa, rowsb)
    isem2 = (isema, isemb)
    gsem2 = (gsema, gsemb)
    ssem2 = (ssema, ssemb)

    def zrow(i, carry):
        for k in range(D // 16):
            rowsa[i, pl.ds(k * 16, 16)] = jnp.zeros((16,), jnp.float32)
        return carry
    lax.fori_loop(0, CH, zrow, 0)

    def zdeg_body(i, carry):
        degv[pl.ds(i * 16, 16)] = jnp.zeros((16,), jnp.float32)
        return carry
    lax.fori_loop(0, SROWS // 16, zdeg_body, 0)

    def ones_body(i, carry):
        ones1d[pl.ds(i * 16, 16)] = jnp.ones((16,), jnp.float32)
        return carry
    lax.fori_loop(0, CH // 16, ones_body, 0)

    def zero_acc_s(q, carry):
        pltpu.sync_copy(rowsa, acc_s.at[pl.ds(s * SROWS + q * CH, CH)])
        return carry
    lax.fori_loop(0, SROWS // CH, zero_acc_s, 0)

    pltpu.sync_copy(rowsa.at[pl.ds(0, DROWS)],
                    acc_d.at[pl.ds(s * DROWS, DROWS)])
    pltpu.sync_copy(degv, deg_s.at[pl.ds(s * SROWS, SROWS)])
    pltpu.sync_copy(degv.at[pl.ds(0, DROWS)],
                    deg_d.at[pl.ds(s * DROWS, DROWS)])

    plsc.subcore_barrier()

    @pl.when(c == 0)
    def _():
        _edge_loop(E_NEAR // NT // CH, s * (E_NEAR // NT), near_src, near_dst,
                   t_near, acc_s, deg_s, sidx2, didx2, rows2, isem2, gsem2,
                   ssem2, ones1d)

    @pl.when(c == 1)
    def _():
        _edge_loop(E_HAS // NT // CH, s * (E_HAS // NT), has_src, has_dst,
                   t_has, acc_s, deg_s, sidx2, didx2, rows2, isem2, gsem2,
                   ssem2, ones1d)
        _edge_loop(E_IN // NT // CH, s * (E_IN // NT), in_src, in_dst,
                   t_in, acc_d, deg_d, sidx2, didx2, rows2, isem2, gsem2,
                   ssem2, ones1d)

    plsc.subcore_barrier()

    @pl.when(c == 0)
    def _():
        _norm_out(SROWS // CH, s * SROWS, acc_s, deg_s, out_near, rowsa, degv,
                  CH)

    @pl.when(c == 1)
    def _():
        _norm_out(SROWS // CH, s * SROWS, acc_s, deg_s, out_has, rowsa, degv,
                  CH)
        _norm_out(1, s * DROWS, acc_d, deg_d, out_in, rowsa, degv, DROWS)


def _make_sc_agg():
    mesh = plsc.VectorSubcoreMesh(core_axis_name="c", subcore_axis_name="s",
                                  num_cores=NC, num_subcores=NT)
    return pl.kernel(
        _sc_agg_body,
        out_type=[jax.ShapeDtypeStruct((NS_PAD, D), jnp.float32),
                  jax.ShapeDtypeStruct((ND_PAD, D), jnp.float32),
                  jax.ShapeDtypeStruct((NS_PAD, D), jnp.float32)],
        mesh=mesh,
        scratch_types=[
            pltpu.VMEM((CH,), jnp.int32),
            pltpu.VMEM((CH,), jnp.int32),
            pltpu.VMEM((CH,), jnp.int32),
            pltpu.VMEM((CH,), jnp.int32),
            pltpu.VMEM((CH, D), jnp.float32),
            pltpu.VMEM((CH, D), jnp.float32),
            pltpu.VMEM((CH,), jnp.float32),
            pltpu.VMEM((SROWS,), jnp.float32),
            pltpu.SemaphoreType.DMA,
            pltpu.SemaphoreType.DMA,
            pltpu.SemaphoreType.DMA,
            pltpu.SemaphoreType.DMA,
            pltpu.SemaphoreType.DMA,
            pltpu.SemaphoreType.DMA,
            pltpu.VMEM_SHARED((NS_PAD, D), jnp.float32),
            pltpu.VMEM_SHARED((ND_PAD, D), jnp.float32),
            pltpu.VMEM_SHARED((NS_PAD,), jnp.float32),
            pltpu.VMEM_SHARED((ND_PAD,), jnp.float32),
        ],
    )


_SC_AGG_CACHE = []


def _sc_agg(*args):
    if not _SC_AGG_CACHE:
        _SC_AGG_CACHE.append(_make_sc_agg())
    return _SC_AGG_CACHE[0](*args)


def _combine_sent_body(agg_n_ref, agg_h_ref, self_ref, bias_ref, w_ref,
                       t_a_ref, t_b_ref, self2_ref):
    h = agg_n_ref[...] + agg_h_ref[...] + self_ref[...] + bias_ref[...]
    h = jnp.maximum(h, 0.0)
    proj = jnp.dot(h, w_ref[...], preferred_element_type=jnp.float32)
    t_a_ref[...] = proj[:, :D]
    t_b_ref[...] = proj[:, D:2 * D]
    self2_ref[...] = proj[:, 2 * D:3 * D]


def _combine_sent(agg_n, agg_h, self_s, bias, w):
    nblk = NS_PAD // SENT_BLK
    return pl.pallas_call(
        _combine_sent_body,
        grid=(nblk,),
        in_specs=[pl.BlockSpec((SENT_BLK, D), lambda i: (i, 0)),
                  pl.BlockSpec((SENT_BLK, D), lambda i: (i, 0)),
                  pl.BlockSpec((SENT_BLK, D), lambda i: (i, 0)),
                  pl.BlockSpec((1, D), lambda i: (0, 0)),
                  pl.BlockSpec((D, 3 * D), lambda i: (0, 0))],
        out_specs=[pl.BlockSpec((SENT_BLK, D), lambda i: (i, 0)),
                   pl.BlockSpec((SENT_BLK, D), lambda i: (i, 0)),
                   pl.BlockSpec((SENT_BLK, D), lambda i: (i, 0))],
        out_shape=[jax.ShapeDtypeStruct((NS_PAD, D), jnp.float32),
                   jax.ShapeDtypeStruct((NS_PAD, D), jnp.float32),
                   jax.ShapeDtypeStruct((NS_PAD, D), jnp.float32)],
    )(agg_n, agg_h, self_s, bias, w)


def _combine_doc_body(agg_ref, self_ref, bias_ref, w_ref, t_ref, self2_ref):
    h = agg_ref[...] + self_ref[...] + bias_ref[...]
    h = jnp.maximum(h, 0.0)
    proj = jnp.dot(h, w_ref[...], preferred_element_type=jnp.float32)
    t_ref[...] = proj[:, :D]
    self2_ref[...] = proj[:, D:2 * D]


def _combine_doc(agg, self_d, bias, w):
    return pl.pallas_call(
        _combine_doc_body,
        grid=(ND_PAD // DOC_BLK,),
        in_specs=[pl.BlockSpec((DOC_BLK, D), lambda i: (i, 0)),
                  pl.BlockSpec((DOC_BLK, D), lambda i: (i, 0)),
                  pl.BlockSpec((1, D), lambda i: (0, 0)),
                  pl.BlockSpec((D, 2 * D), lambda i: (0, 0))],
        out_specs=[pl.BlockSpec((DOC_BLK, D), lambda i: (i, 0)),
                   pl.BlockSpec((DOC_BLK, D), lambda i: (i, 0))],
        out_shape=[jax.ShapeDtypeStruct((ND_PAD, D), jnp.float32),
                   jax.ShapeDtypeStruct((ND_PAD, D), jnp.float32)],
    )(agg, self_d, bias, w)


def _reduce_sent_body(agg_n_ref, agg_h_ref, self_ref, bias_ref, out_ref):
    i = pl.program_id(0)
    h = agg_n_ref[...] + agg_h_ref[...] + self_ref[...] + bias_ref[...]
    h = jnp.maximum(h, 0.0)
    row = lax.broadcasted_iota(jnp.int32, h.shape, 0) + i * SENT_BLK
    h = jnp.where(row < N_SENT, h, 0.0)
    part = jnp.sum(h, axis=0, keepdims=True)

    @pl.when(i == 0)
    def _():
        out_ref[...] = part

    @pl.when(i > 0)
    def _():
        out_ref[...] += part


def _reduce_sent(agg_n, agg_h, self_s, bias):
    nblk = NS_PAD // SENT_BLK
    return pl.pallas_call(
        _reduce_sent_body,
        grid=(nblk,),
        in_specs=[pl.BlockSpec((SENT_BLK, D), lambda i: (i, 0)),
                  pl.BlockSpec((SENT_BLK, D), lambda i: (i, 0)),
                  pl.BlockSpec((SENT_BLK, D), lambda i: (i, 0)),
                  pl.BlockSpec((1, D), lambda i: (0, 0))],
        out_specs=pl.BlockSpec((1, D), lambda i: (0, 0)),
        out_shape=jax.ShapeDtypeStruct((1, D), jnp.float32),
    )(agg_n, agg_h, self_s, bias)


def _reduce_doc_body(agg_ref, self_ref, bias_ref, out_ref):
    h = agg_ref[...] + self_ref[...] + bias_ref[...]
    h = jnp.maximum(h, 0.0)
    row = lax.broadcasted_iota(jnp.int32, h.shape, 0)
    h = jnp.where(row < N_DOC, h, 0.0)
    out_ref[...] = jnp.sum(h, axis=0, keepdims=True)


def _reduce_doc(agg, self_d, bias):
    return pl.pallas_call(
        _reduce_doc_body,
        grid=(ND_PAD // DOC_BLK,),
        in_specs=[pl.BlockSpec((DOC_BLK, D), lambda i: (i, 0)),
                  pl.BlockSpec((DOC_BLK, D), lambda i: (i, 0)),
                  pl.BlockSpec((1, D), lambda i: (0, 0))],
        out_specs=pl.BlockSpec((1, D), lambda i: (0, 0)),
        out_shape=jax.ShapeDtypeStruct((1, D), jnp.float32),
    )(agg, self_d, bias)


def _final_body(ssum_ref, dsum_ref, w_ref, b_ref, out_ref):
    total = ssum_ref[...] + dsum_ref[...]
    out_ref[...] = (jnp.dot(total, w_ref[...],
                            preferred_element_type=jnp.float32)
                    + b_ref[...])


def _final(ssum, dsum, scorer_w, scorer_b):
    return pl.pallas_call(
        _final_body,
        in_specs=[pl.BlockSpec((1, D), lambda: (0, 0)),
                  pl.BlockSpec((1, D), lambda: (0, 0)),
                  pl.BlockSpec((D, 1), lambda: (0, 0)),
                  pl.BlockSpec((1, 1), lambda: (0, 0))],
        out_specs=pl.BlockSpec((1, 1), lambda: (0, 0)),
        out_shape=jax.ShapeDtypeStruct((1, 1), jnp.float32),
    )(ssum, dsum, scorer_w, scorer_b)


def kernel(sent_feat, doc_feat, near_src, near_dst, in_src, in_dst, has_src,
           has_dst, basis1, coeff1, h_bias1, loop_w1, basis2, coeff2, h_bias2,
           loop_w2, scorer_w, scorer_b):
    w_s1, w_d1, w_s2, w_d2 = _prep_weights(coeff1, coeff2, basis1, loop_w1,
                                           basis2, loop_w2)
    bias1 = h_bias1.reshape(1, D)
    bias2 = h_bias2.reshape(1, D)
    sent_p = jnp.pad(sent_feat, ((0, NS_PAD - N_SENT), (0, 0)))
    doc_p = jnp.pad(doc_feat, ((0, ND_PAD - N_DOC), (0, 0)))

    t_near, t_in, self_s = _proj_sent(sent_p, w_s1, D_IN)
    t_has, self_d = _proj_doc(doc_p, w_d1, D_IN)

    agg_n, agg_i, agg_h = _sc_agg(t_near, t_in, t_has, near_src, near_dst,
                                  in_src, in_dst, has_src, has_dst)

    t2_near, t2_in, self2_s = _combine_sent(agg_n, agg_h, self_s, bias1, w_s2)
    t2_has, self2_d = _combine_doc(agg_i, self_d, bias1, w_d2)

    agg2_n, agg2_i, agg2_h = _sc_agg(t2_near, t2_in, t2_has, near_src,
                                     near_dst, in_src, in_dst, has_src,
                                     has_dst)

    ssum = _reduce_sent(agg2_n, agg2_h, self2_s, bias2)
    dsum = _reduce_doc(agg2_i, self2_d, bias2)
    return _final(ssum, dsum, scorer_w, scorer_b.reshape(1, 1))


# speedup vs baseline: 1.0441x; 1.0441x over previous
import functools

import jax
import jax.numpy as jnp
from jax import lax
from jax.experimental import pallas as pl
from jax.experimental.pallas import tpu as pltpu
from jax.experimental.pallas import tpu_sc as plsc

N_SENT = 10000
N_DOC = 1000
NS_PAD = 10240
ND_PAD = 1024
D_IN = 768
D = 128
E_NEAR = 51200
E_IN = 25600
E_HAS = 25600

NC = 2
NT = 16
CH = 80

SENT_BLK = 512
DOC_BLK = 1024

SROWS = NS_PAD // NT
DROWS = ND_PAD // NT
ZCH = 64


def _prep_body(coeff1_ref, coeff2_ref, basis1_ref, loop1_ref, basis2_ref,
               loop2_ref, w_s1_ref, w_d1_ref, w_s2_ref, w_d2_ref):
    b1a = basis1_ref[0]
    b1b = basis1_ref[1]
    w10 = coeff1_ref[0, 0] * b1a + coeff1_ref[0, 1] * b1b
    w11 = coeff1_ref[1, 0] * b1a + coeff1_ref[1, 1] * b1b
    w12 = coeff1_ref[2, 0] * b1a + coeff1_ref[2, 1] * b1b
    w_s1_ref[...] = jnp.concatenate([w10, w11, loop1_ref[...]], axis=1)
    w_d1_ref[...] = jnp.concatenate([w12, loop1_ref[...]], axis=1)
    b2a = basis2_ref[0]
    b2b = basis2_ref[1]
    w20 = coeff2_ref[0, 0] * b2a + coeff2_ref[0, 1] * b2b
    w21 = coeff2_ref[1, 0] * b2a + coeff2_ref[1, 1] * b2b
    w22 = coeff2_ref[2, 0] * b2a + coeff2_ref[2, 1] * b2b
    w_s2_ref[...] = jnp.concatenate([w20, w21, loop2_ref[...]], axis=1)
    w_d2_ref[...] = jnp.concatenate([w22, loop2_ref[...]], axis=1)


def _prep_weights(coeff1, coeff2, basis1, loop_w1, basis2, loop_w2):
    smem = pl.BlockSpec(memory_space=pltpu.MemorySpace.SMEM)
    return pl.pallas_call(
        _prep_body,
        in_specs=[smem, smem, pl.BlockSpec((2, D_IN, D), lambda: (0, 0, 0)),
                  pl.BlockSpec((D_IN, D), lambda: (0, 0)),
                  pl.BlockSpec((2, D, D), lambda: (0, 0, 0)),
                  pl.BlockSpec((D, D), lambda: (0, 0))],
        out_specs=[pl.BlockSpec((D_IN, 3 * D), lambda: (0, 0)),
                   pl.BlockSpec((D_IN, 2 * D), lambda: (0, 0)),
                   pl.BlockSpec((D, 3 * D), lambda: (0, 0)),
                   pl.BlockSpec((D, 2 * D), lambda: (0, 0))],
        out_shape=[jax.ShapeDtypeStruct((D_IN, 3 * D), jnp.float32),
                   jax.ShapeDtypeStruct((D_IN, 2 * D), jnp.float32),
                   jax.ShapeDtypeStruct((D, 3 * D), jnp.float32),
                   jax.ShapeDtypeStruct((D, 2 * D), jnp.float32)],
    )(coeff1, coeff2, basis1, loop_w1, basis2, loop_w2)


def _proj_sent_body(x_ref, w_ref, t_a_ref, t_b_ref, self_ref):
    proj = jnp.dot(x_ref[...], w_ref[...], preferred_element_type=jnp.float32)
    t_a_ref[...] = proj[:, :D]
    t_b_ref[...] = proj[:, D:2 * D]
    self_ref[...] = proj[:, 2 * D:3 * D]


def _proj_sent(x, w, d_in):
    nblk = NS_PAD // SENT_BLK
    return pl.pallas_call(
        _proj_sent_body,
        grid=(nblk,),
        in_specs=[pl.BlockSpec((SENT_BLK, d_in), lambda i: (i, 0)),
                  pl.BlockSpec((d_in, 3 * D), lambda i: (0, 0))],
        out_specs=[pl.BlockSpec((SENT_BLK, D), lambda i: (i, 0)),
                   pl.BlockSpec((SENT_BLK, D), lambda i: (i, 0)),
                   pl.BlockSpec((SENT_BLK, D), lambda i: (i, 0))],
        out_shape=[jax.ShapeDtypeStruct((NS_PAD, D), jnp.float32),
                   jax.ShapeDtypeStruct((NS_PAD, D), jnp.float32),
                   jax.ShapeDtypeStruct((NS_PAD, D), jnp.float32)],
    )(x, w)


def _proj_doc_body(x_ref, w_ref, t_ref, self_ref):
    proj = jnp.dot(x_ref[...], w_ref[...], preferred_element_type=jnp.float32)
    t_ref[...] = proj[:, :D]
    self_ref[...] = proj[:, D:2 * D]


def _proj_doc(x, w, d_in):
    return pl.pallas_call(
        _proj_doc_body,
        grid=(ND_PAD // DOC_BLK,),
        in_specs=[pl.BlockSpec((DOC_BLK, d_in), lambda i: (i, 0)),
                  pl.BlockSpec((d_in, 2 * D), lambda i: (0, 0))],
        out_specs=[pl.BlockSpec((DOC_BLK, D), lambda i: (i, 0)),
                   pl.BlockSpec((DOC_BLK, D), lambda i: (i, 0))],
        out_shape=[jax.ShapeDtypeStruct((ND_PAD, D), jnp.float32),
                   jax.ShapeDtypeStruct((ND_PAD, D), jnp.float32)],
    )(x, w)


def _edge_loop(nchunks, tile_base, src_hbm, dst_hbm, table_hbm, acc_sh,
               deg_sh, sidx2, didx2, rows2, isem2, gsem2, ones1d):
    def issue_idx(j, p):
        base = tile_base + j * CH
        pltpu.async_copy(src_hbm.at[pl.ds(base, CH)], sidx2[p], isem2[p])
        pltpu.async_copy(dst_hbm.at[pl.ds(base, CH)], didx2[p], isem2[p])

    def wait_idx(p):
        pltpu.make_async_copy(src_hbm.at[pl.ds(0, CH)], sidx2[p],
                              isem2[p]).wait()
        pltpu.make_async_copy(dst_hbm.at[pl.ds(0, CH)], didx2[p],
                              isem2[p]).wait()

    def issue_gather(p):
        pltpu.async_copy(table_hbm.at[sidx2[p]], rows2[p], gsem2[p])

    def wait_gather(p):
        pltpu.make_async_copy(table_hbm.at[sidx2[p]], rows2[p],
                              gsem2[p]).wait()

    def scatter(p):
        pltpu.sync_copy(rows2[p], acc_sh.at[didx2[p]], add=True)
        if deg_sh is not None:
            pltpu.sync_copy(ones1d, deg_sh.at[didx2[p]], add=True)

    nk = nchunks // 2
    issue_idx(0, 0)

    def body(k, carry):
        j0 = 2 * k
        wait_idx(0)
        issue_gather(0)
        issue_idx(j0 + 1, 1)
        wait_idx(1)
        issue_gather(1)
        wait_gather(0)
        scatter(0)

        @pl.when(k < nk - 1)
        def _():
            issue_idx(j0 + 2, 0)
        wait_gather(1)
        scatter(1)
        return carry
    lax.fori_loop(0, nk, body, 0)


def _norm_out(nchunks, row0, acc_sh, deg_sh, out_hbm, nbuf, degv, chrows):
    pltpu.sync_copy(deg_sh.at[pl.ds(row0, nchunks * chrows)],
                    degv.at[pl.ds(0, nchunks * chrows)])
    for q in range(nchunks):
        pltpu.sync_copy(acc_sh.at[pl.ds(row0 + q * chrows, chrows)],
                        nbuf.at[pl.ds(0, chrows)])

        def grp_body(g, carry):
            deg16 = degv[pl.ds(q * chrows + g * 16, 16)]
            rd16 = 1.0 / jnp.maximum(deg16, 1.0)
            for r in range(16):
                row = g * 16 + r
                rd = rd16[r]
                for k in range(D // 16):
                    nbuf[row, pl.ds(k * 16, 16)] = (
                        nbuf[row, pl.ds(k * 16, 16)] * rd)
            return carry
        lax.fori_loop(0, chrows // 16, grp_body, 0)
        pltpu.sync_copy(nbuf.at[pl.ds(0, chrows)],
                        out_hbm.at[pl.ds(row0 + q * chrows, chrows)])


def _sc_agg_body(with_deg, *refs):
    if with_deg:
        (t_near, t_in, t_has, near_src, near_dst, in_src, in_dst,
         has_src, has_dst, out_near, out_in, out_has, dn_out, dh_out, di_out,
         sidxa, sidxb, didxa, didxb, rowsa, rowsb, ones1d, degv,
         isema, isemb, gsema, gsemb, acc_s, acc_d, deg_s, deg_d) = refs
    else:
        (t_near, t_in, t_has, near_src, near_dst, in_src, in_dst,
         has_src, has_dst, dn_in, dh_in, di_in,
         out_near, out_in, out_has,
         sidxa, sidxb, didxa, didxb, rowsa, rowsb, ones1d, degv,
         isema, isemb, gsema, gsemb, acc_s, acc_d) = refs
        deg_s = deg_d = None
    c = lax.axis_index("c")
    s = lax.axis_index("s")
    sidx2 = (sidxa, sidxb)
    didx2 = (didxa, didxb)
    rows2 = (rowsa, rowsb)
    isem2 = (isema, isemb)
    gsem2 = (gsema, gsemb)

    def zrow(i, carry):
        for k in range(D // 16):
            rowsa[i, pl.ds(k * 16, 16)] = jnp.zeros((16,), jnp.float32)
        return carry
    lax.fori_loop(0, CH, zrow, 0)

    def zero_acc_s(q, carry):
        pltpu.sync_copy(rowsa, acc_s.at[pl.ds(s * SROWS + q * CH, CH)])
        return carry
    lax.fori_loop(0, SROWS // CH, zero_acc_s, 0)

    pltpu.sync_copy(rowsa.at[pl.ds(0, DROWS)],
                    acc_d.at[pl.ds(s * DROWS, DROWS)])

    if with_deg:
        def zdeg_body(i, carry):
            degv[pl.ds(i * 16, 16)] = jnp.zeros((16,), jnp.float32)
            return carry
        lax.fori_loop(0, SROWS // 16, zdeg_body, 0)

        def ones_body(i, carry):
            ones1d[pl.ds(i * 16, 16)] = jnp.ones((16,), jnp.float32)
            return carry
        lax.fori_loop(0, CH // 16, ones_body, 0)

        pltpu.sync_copy(degv, deg_s.at[pl.ds(s * SROWS, SROWS)])
        pltpu.sync_copy(degv.at[pl.ds(0, DROWS)],
                        deg_d.at[pl.ds(s * DROWS, DROWS)])

    plsc.subcore_barrier()

    @pl.when(c == 0)
    def _():
        _edge_loop(E_NEAR // NT // CH, s * (E_NEAR // NT), near_src, near_dst,
                   t_near, acc_s, deg_s, sidx2, didx2, rows2, isem2, gsem2,
                   ones1d)

    @pl.when(c == 1)
    def _():
        _edge_loop(E_HAS // NT // CH, s * (E_HAS // NT), has_src, has_dst,
                   t_has, acc_s, deg_s, sidx2, didx2, rows2, isem2, gsem2,
                   ones1d)
        _edge_loop(E_IN // NT // CH, s * (E_IN // NT), in_src, in_dst,
                   t_in, acc_d, deg_d, sidx2, didx2, rows2, isem2, gsem2,
                   ones1d)

    plsc.subcore_barrier()

    deg_near_src = deg_s if with_deg else dn_in
    deg_has_src = deg_s if with_deg else dh_in
    deg_in_src = deg_d if with_deg else di_in

    @pl.when(c == 0)
    def _():
        _norm_out(SROWS // CH, s * SROWS, acc_s, deg_near_src, out_near,
                  rowsa, degv, CH)
        if with_deg:
            pltpu.sync_copy(degv, dn_out.at[pl.ds(s * SROWS, SROWS)])

    @pl.when(c == 1)
    def _():
        _norm_out(SROWS // CH, s * SROWS, acc_s, deg_has_src, out_has,
                  rowsa, degv, CH)
        if with_deg:
            pltpu.sync_copy(degv, dh_out.at[pl.ds(s * SROWS, SROWS)])
        _norm_out(1, s * DROWS, acc_d, deg_in_src, out_in, rowsa, degv, DROWS)
        if with_deg:
            pltpu.sync_copy(degv.at[pl.ds(0, DROWS)],
                            di_out.at[pl.ds(s * DROWS, DROWS)])


def _make_sc_agg(with_deg):
    mesh = plsc.VectorSubcoreMesh(core_axis_name="c", subcore_axis_name="s",
                                  num_cores=NC, num_subcores=NT)
    out_type = [jax.ShapeDtypeStruct((NS_PAD, D), jnp.float32),
                jax.ShapeDtypeStruct((ND_PAD, D), jnp.float32),
                jax.ShapeDtypeStruct((NS_PAD, D), jnp.float32)]
    if with_deg:
        out_type += [jax.ShapeDtypeStruct((NS_PAD,), jnp.float32),
                     jax.ShapeDtypeStruct((NS_PAD,), jnp.float32),
                     jax.ShapeDtypeStruct((ND_PAD,), jnp.float32)]
    scratch = [
        pltpu.VMEM((CH,), jnp.int32),
        pltpu.VMEM((CH,), jnp.int32),
        pltpu.VMEM((CH,), jnp.int32),
        pltpu.VMEM((CH,), jnp.int32),
        pltpu.VMEM((CH, D), jnp.float32),
        pltpu.VMEM((CH, D), jnp.float32),
        pltpu.VMEM((CH,), jnp.float32),
        pltpu.VMEM((SROWS,), jnp.float32),
        pltpu.SemaphoreType.DMA,
        pltpu.SemaphoreType.DMA,
        pltpu.SemaphoreType.DMA,
        pltpu.SemaphoreType.DMA,
        pltpu.VMEM_SHARED((NS_PAD, D), jnp.float32),
        pltpu.VMEM_SHARED((ND_PAD, D), jnp.float32),
    ]
    if with_deg:
        scratch += [
            pltpu.VMEM_SHARED((NS_PAD,), jnp.float32),
            pltpu.VMEM_SHARED((ND_PAD,), jnp.float32),
        ]
    return pl.kernel(
        functools.partial(_sc_agg_body, with_deg),
        out_type=out_type,
        mesh=mesh,
        scratch_types=scratch,
    )


_SC_AGG_CACHE = {}


def _sc_agg(with_deg, *args):
    if with_deg not in _SC_AGG_CACHE:
        _SC_AGG_CACHE[with_deg] = _make_sc_agg(with_deg)
    return _SC_AGG_CACHE[with_deg](*args)


def _combine_sent_body(agg_n_ref, agg_h_ref, self_ref, bias_ref, w_ref,
                       t_a_ref, t_b_ref, self2_ref):
    h = agg_n_ref[...] + agg_h_ref[...] + self_ref[...] + bias_ref[...]
    h = jnp.maximum(h, 0.0)
    proj = jnp.dot(h, w_ref[...], preferred_element_type=jnp.float32)
    t_a_ref[...] = proj[:, :D]
    t_b_ref[...] = proj[:, D:2 * D]
    self2_ref[...] = proj[:, 2 * D:3 * D]


def _combine_sent(agg_n, agg_h, self_s, bias, w):
    nblk = NS_PAD // SENT_BLK
    return pl.pallas_call(
        _combine_sent_body,
        grid=(nblk,),
        in_specs=[pl.BlockSpec((SENT_BLK, D), lambda i: (i, 0)),
                  pl.BlockSpec((SENT_BLK, D), lambda i: (i, 0)),
                  pl.BlockSpec((SENT_BLK, D), lambda i: (i, 0)),
                  pl.BlockSpec((1, D), lambda i: (0, 0)),
                  pl.BlockSpec((D, 3 * D), lambda i: (0, 0))],
        out_specs=[pl.BlockSpec((SENT_BLK, D), lambda i: (i, 0)),
                   pl.BlockSpec((SENT_BLK, D), lambda i: (i, 0)),
                   pl.BlockSpec((SENT_BLK, D), lambda i: (i, 0))],
        out_shape=[jax.ShapeDtypeStruct((NS_PAD, D), jnp.float32),
                   jax.ShapeDtypeStruct((NS_PAD, D), jnp.float32),
                   jax.ShapeDtypeStruct((NS_PAD, D), jnp.float32)],
    )(agg_n, agg_h, self_s, bias, w)


def _combine_doc_body(agg_ref, self_ref, bias_ref, w_ref, t_ref, self2_ref):
    h = agg_ref[...] + self_ref[...] + bias_ref[...]
    h = jnp.maximum(h, 0.0)
    proj = jnp.dot(h, w_ref[...], preferred_element_type=jnp.float32)
    t_ref[...] = proj[:, :D]
    self2_ref[...] = proj[:, D:2 * D]


def _combine_doc(agg, self_d, bias, w):
    return pl.pallas_call(
        _combine_doc_body,
        grid=(ND_PAD // DOC_BLK,),
        in_specs=[pl.BlockSpec((DOC_BLK, D), lambda i: (i, 0)),
                  pl.BlockSpec((DOC_BLK, D), lambda i: (i, 0)),
                  pl.BlockSpec((1, D), lambda i: (0, 0)),
                  pl.BlockSpec((D, 2 * D), lambda i: (0, 0))],
        out_specs=[pl.BlockSpec((DOC_BLK, D), lambda i: (i, 0)),
                   pl.BlockSpec((DOC_BLK, D), lambda i: (i, 0))],
        out_shape=[jax.ShapeDtypeStruct((ND_PAD, D), jnp.float32),
                   jax.ShapeDtypeStruct((ND_PAD, D), jnp.float32)],
    )(agg, self_d, bias, w)


def _reduce_sent_body(agg_n_ref, agg_h_ref, self_ref, bias_ref, out_ref):
    i = pl.program_id(0)
    h = agg_n_ref[...] + agg_h_ref[...] + self_ref[...] + bias_ref[...]
    h = jnp.maximum(h, 0.0)
    row = lax.broadcasted_iota(jnp.int32, h.shape, 0) + i * SENT_BLK
    h = jnp.where(row < N_SENT, h, 0.0)
    part = jnp.sum(h, axis=0, keepdims=True)

    @pl.when(i == 0)
    def _():
        out_ref[...] = part

    @pl.when(i > 0)
    def _():
        out_ref[...] += part


def _reduce_sent(agg_n, agg_h, self_s, bias):
    nblk = NS_PAD // SENT_BLK
    return pl.pallas_call(
        _reduce_sent_body,
        grid=(nblk,),
        in_specs=[pl.BlockSpec((SENT_BLK, D), lambda i: (i, 0)),
                  pl.BlockSpec((SENT_BLK, D), lambda i: (i, 0)),
                  pl.BlockSpec((SENT_BLK, D), lambda i: (i, 0)),
                  pl.BlockSpec((1, D), lambda i: (0, 0))],
        out_specs=pl.BlockSpec((1, D), lambda i: (0, 0)),
        out_shape=jax.ShapeDtypeStruct((1, D), jnp.float32),
    )(agg_n, agg_h, self_s, bias)


def _reduce_doc_body(agg_ref, self_ref, bias_ref, out_ref):
    h = agg_ref[...] + self_ref[...] + bias_ref[...]
    h = jnp.maximum(h, 0.0)
    row = lax.broadcasted_iota(jnp.int32, h.shape, 0)
    h = jnp.where(row < N_DOC, h, 0.0)
    out_ref[...] = jnp.sum(h, axis=0, keepdims=True)


def _reduce_doc(agg, self_d, bias):
    return pl.pallas_call(
        _reduce_doc_body,
        grid=(ND_PAD // DOC_BLK,),
        in_specs=[pl.BlockSpec((DOC_BLK, D), lambda i: (i, 0)),
                  pl.BlockSpec((DOC_BLK, D), lambda i: (i, 0)),
                  pl.BlockSpec((1, D), lambda i: (0, 0))],
        out_specs=pl.BlockSpec((1, D), lambda i: (0, 0)),
        out_shape=jax.ShapeDtypeStruct((1, D), jnp.float32),
    )(agg, self_d, bias)


def _final_body(ssum_ref, dsum_ref, w_ref, b_ref, out_ref):
    total = ssum_ref[...] + dsum_ref[...]
    out_ref[...] = (jnp.dot(total, w_ref[...],
                            preferred_element_type=jnp.float32)
                    + b_ref[...])


def _final(ssum, dsum, scorer_w, scorer_b):
    return pl.pallas_call(
        _final_body,
        in_specs=[pl.BlockSpec((1, D), lambda: (0, 0)),
                  pl.BlockSpec((1, D), lambda: (0, 0)),
                  pl.BlockSpec((D, 1), lambda: (0, 0)),
                  pl.BlockSpec((1, 1), lambda: (0, 0))],
        out_specs=pl.BlockSpec((1, 1), lambda: (0, 0)),
        out_shape=jax.ShapeDtypeStruct((1, 1), jnp.float32),
    )(ssum, dsum, scorer_w, scorer_b)


def kernel(sent_feat, doc_feat, near_src, near_dst, in_src, in_dst, has_src,
           has_dst, basis1, coeff1, h_bias1, loop_w1, basis2, coeff2, h_bias2,
           loop_w2, scorer_w, scorer_b):
    w_s1, w_d1, w_s2, w_d2 = _prep_weights(coeff1, coeff2, basis1, loop_w1,
                                           basis2, loop_w2)
    bias1 = h_bias1.reshape(1, D)
    bias2 = h_bias2.reshape(1, D)
    sent_p = jnp.pad(sent_feat, ((0, NS_PAD - N_SENT), (0, 0)))
    doc_p = jnp.pad(doc_feat, ((0, ND_PAD - N_DOC), (0, 0)))

    t_near, t_in, self_s = _proj_sent(sent_p, w_s1, D_IN)
    t_has, self_d = _proj_doc(doc_p, w_d1, D_IN)

    agg_n, agg_i, agg_h, deg_n, deg_h, deg_i = _sc_agg(
        True, t_near, t_in, t_has, near_src, near_dst, in_src, in_dst,
        has_src, has_dst)

    t2_near, t2_in, self2_s = _combine_sent(agg_n, agg_h, self_s, bias1, w_s2)
    t2_has, self2_d = _combine_doc(agg_i, self_d, bias1, w_d2)

    agg2_n, agg2_i, agg2_h = _sc_agg(
        False, t2_near, t2_in, t2_has, near_src, near_dst, in_src, in_dst,
        has_src, has_dst, deg_n, deg_h, deg_i)

    ssum = _reduce_sent(agg2_n, agg2_h, self2_s, bias2)
    dsum = _reduce_doc(agg2_i, self2_d, bias2)
    return _final(ssum, dsum, scorer_w, scorer_b.reshape(1, 1))
